# baseline (speedup 1.0000x reference)
import functools

import jax
import jax.numpy as jnp
from jax.experimental import pallas as pl


def _square_distance(src, dst):
    return (jnp.sum(src ** 2, -1)[:, :, None]
            + jnp.sum(dst ** 2, -1)[:, None, :]
            - 2.0 * jnp.einsum('bnc,bmc->bnm', src, dst))


def _pd_kernel(src_ref, dstT_ref, o_ref, *, radius, n_sentinel):
    a = src_ref[0]
    xT = dstT_ref[0]
    s2 = (a[:, 0:1] * a[:, 0:1] + a[:, 1:2] * a[:, 1:2]) + a[:, 2:3] * a[:, 2:3]
    x2 = (xT[0:1] * xT[0:1] + xT[1:2] * xT[1:2]) + xT[2:3] * xT[2:3]
    e = jax.lax.dot_general(
        a.astype(jnp.bfloat16), xT.astype(jnp.bfloat16),
        (((1,), (0,)), ((), ())), preferred_element_type=jnp.float32)
    d2 = (s2 + x2) - 2.0 * e
    if radius is None:
        o_ref[0] = d2
    else:
        iota = jax.lax.broadcasted_iota(jnp.int32, d2.shape, 1)
        o_ref[0] = jnp.where(d2 > radius ** 2, n_sentinel, iota)


def _pairdist(src, dst, radius=None):
    b, s, _ = src.shape
    n = dst.shape[1]
    dst_t = jnp.transpose(dst, (0, 2, 1))
    r = min(s, 256)
    out_dtype = jnp.float32 if radius is None else jnp.int32
    return pl.pallas_call(
        functools.partial(_pd_kernel, radius=radius, n_sentinel=n),
        grid=(b, s // r),
        in_specs=[
            pl.BlockSpec((1, r, 3), lambda i, j: (i, j, 0)),
            pl.BlockSpec((1, 3, n), lambda i, j: (i, 0, 0)),
        ],
        out_specs=pl.BlockSpec((1, r, n), lambda i, j: (i, j, 0)),
        out_shape=jax.ShapeDtypeStruct((b, s, n), out_dtype),
    )(src, dst_t)


def _index_points(points, idx):
    return jax.vmap(lambda p, i: p[i])(points, idx)


def _fps_kernel(x_ref, y_ref, z_ref, out_ref, *, npoint, n, b):
    x = x_ref[...]
    y = y_ref[...]
    z = z_ref[...]
    iota_n = jax.lax.broadcasted_iota(jnp.int32, (b, n), 1)
    iota_s = jax.lax.broadcasted_iota(jnp.int32, (b, npoint), 1)

    out_ref[...] = jnp.zeros((b, npoint), jnp.int32)

    def body(i, state):
        distance, farthest = state
        far_s = jnp.broadcast_to(farthest, (b, npoint))
        out_ref[...] = jnp.where(iota_s == i, far_s, out_ref[...])
        sel = iota_n == jnp.broadcast_to(farthest, (b, n))
        cx = jnp.sum(jnp.where(sel, x, 0.0), axis=1, keepdims=True)
        cy = jnp.sum(jnp.where(sel, y, 0.0), axis=1, keepdims=True)
        cz = jnp.sum(jnp.where(sel, z, 0.0), axis=1, keepdims=True)
        d = (x - cx) ** 2 + (y - cy) ** 2 + (z - cz) ** 2
        distance = jnp.minimum(distance, d)
        m = jnp.max(distance, axis=1, keepdims=True)
        farthest = jnp.min(jnp.where(distance == m, iota_n, n), axis=1,
                           keepdims=True).astype(jnp.int32)
        return distance, farthest

    dist0 = jnp.full((b, n), 1e10, jnp.float32)
    far0 = jnp.zeros((b, 1), jnp.int32)
    jax.lax.fori_loop(0, npoint, body, (dist0, far0))


def _farthest_point_sample(xyz, npoint):
    xyz = jax.lax.stop_gradient(xyz)
    b, n, _ = xyz.shape

    def body(i, state):
        centroids, distance, farthest = state
        centroids = centroids.at[:, i].set(farthest)
        centroid = _index_points(xyz, farthest[:, None])
        dist = jnp.sum((xyz - centroid) ** 2, -1)
        distance = jnp.minimum(distance, dist)
        farthest = jnp.argmax(distance, -1).astype(jnp.int32)
        return (centroids, distance, farthest)

    centroids = jnp.zeros((b, npoint), dtype=jnp.int32)
    distance = jnp.full((b, n), 1e10, dtype=xyz.dtype)
    farthest = jnp.zeros((b,), dtype=jnp.int32)
    centroids, _, _ = jax.lax.fori_loop(0, npoint, body,
                                        (centroids, distance, farthest))
    return centroids


def _query_ball_point(radius, nsample, xyz, new_xyz):
    xyz = jax.lax.stop_gradient(xyz)
    new_xyz = jax.lax.stop_gradient(new_xyz)
    n = xyz.shape[1]
    sqrdists = _square_distance(new_xyz, xyz)
    group_idx = jnp.broadcast_to(
        jnp.arange(n, dtype=jnp.int32)[None, None, :], sqrdists.shape)
    group_idx = jnp.where(sqrdists > radius ** 2, n, group_idx)
    group_idx = -jax.lax.top_k(-group_idx, nsample)[0]
    group_first = jnp.broadcast_to(group_idx[:, :, :1], group_idx.shape)
    group_idx = jnp.where(group_idx == n, group_first, group_idx)
    return group_idx


def _batchnorm(x, gamma, beta, axes):
    mean = jnp.mean(x, axis=axes, keepdims=True)
    var = jnp.mean((x - mean) ** 2, axis=axes, keepdims=True)
    return gamma * (x - mean) / jnp.sqrt(var + 1e-5) + beta


def _sa_layer(xyz, points, npoint, radius, nsample, layers):
    fps_idx = _farthest_point_sample(xyz, npoint)
    new_xyz = _index_points(xyz, fps_idx)
    idx = _query_ball_point(radius, nsample, xyz, new_xyz)
    grouped_xyz = _index_points(xyz, idx)
    grouped_xyz_norm = grouped_xyz - new_xyz[:, :, None, :]
    grouped_points = _index_points(points, idx)
    h = jnp.concatenate([grouped_xyz_norm, grouped_points], axis=-1)
    for lyr in layers:
        h = h @ lyr['W'].T + lyr['b']
        h = _batchnorm(h, lyr['gamma'], lyr['beta'], (0, 1, 2))
        h = jax.nn.relu(h)
    new_points = jnp.max(h, axis=2)
    return new_xyz, new_points


def _fp_layer(xyz1, xyz2, points1, points2, layers):
    b, n, _ = xyz1.shape
    s = xyz2.shape[1]
    if s == 1:
        interpolated = jnp.broadcast_to(points2, (b, n, points2.shape[-1]))
    else:
        dists = _square_distance(xyz1, xyz2)
        idx = jnp.argsort(dists, axis=-1)[:, :, :3]
        d3 = jnp.take_along_axis(dists, idx, axis=-1)
        dist_recip = 1.0 / (d3 + 1e-8)
        norm = jnp.sum(dist_recip, axis=2, keepdims=True)
        weight = dist_recip / norm
        interpolated = jnp.sum(
            _index_points(points2, idx) * weight[..., None], axis=2)
    if points1 is not None:
        h = jnp.concatenate([points1, interpolated], axis=-1)
    else:
        h = interpolated
    for lyr in layers:
        h = h @ lyr['W'].T + lyr['b']
        h = _batchnorm(h, lyr['gamma'], lyr['beta'], (0, 1))
        h = jax.nn.relu(h)
    return h


_ROWS = 1024


def _mm_bias_kernel(x_ref, w_ref, b_ref, o_ref):
    o_ref[...] = (
        jnp.dot(x_ref[...], w_ref[...],
                preferred_element_type=jnp.float32) + b_ref[...])


def _bn_relu_mm_kernel(y_ref, scale_ref, shift_ref, w_ref, b_ref, o_ref):
    h = jax.nn.relu(y_ref[...] * scale_ref[...] + shift_ref[...])
    o_ref[...] = (
        jnp.dot(h, w_ref[...], preferred_element_type=jnp.float32)
        + b_ref[...])


def _head(h, params):
    b, n, c = h.shape
    x = h.reshape(b * n, c)
    w1t = params['conv1']['W'].T
    b1 = params['conv1']['b'][None, :]
    y1 = pl.pallas_call(
        _mm_bias_kernel,
        grid=(b * n // _ROWS,),
        in_specs=[
            pl.BlockSpec((_ROWS, c), lambda i: (i, 0)),
            pl.BlockSpec((c, c), lambda i: (0, 0)),
            pl.BlockSpec((1, c), lambda i: (0, 0)),
        ],
        out_specs=pl.BlockSpec((_ROWS, c), lambda i: (i, 0)),
        out_shape=jax.ShapeDtypeStruct((b * n, c), jnp.float32),
    )(x, w1t, b1)

    mean = jnp.mean(y1, axis=0, keepdims=True)
    var = jnp.mean((y1 - mean) ** 2, axis=0, keepdims=True)
    gamma = params['bn1']['gamma'][None, :]
    beta = params['bn1']['beta'][None, :]
    scale = gamma / jnp.sqrt(var + 1e-5)
    shift = beta - mean * scale

    w2 = params['conv2']['W']
    nc = w2.shape[0]
    w2t_pad = jnp.zeros((c, 128), jnp.float32).at[:, :nc].set(w2.T)
    b2_pad = jnp.zeros((1, 128), jnp.float32).at[0, :nc].set(
        params['conv2']['b'])
    out = pl.pallas_call(
        _bn_relu_mm_kernel,
        grid=(b * n // _ROWS,),
        in_specs=[
            pl.BlockSpec((_ROWS, c), lambda i: (i, 0)),
            pl.BlockSpec((1, c), lambda i: (0, 0)),
            pl.BlockSpec((1, c), lambda i: (0, 0)),
            pl.BlockSpec((c, 128), lambda i: (0, 0)),
            pl.BlockSpec((1, 128), lambda i: (0, 0)),
        ],
        out_specs=pl.BlockSpec((_ROWS, 128), lambda i: (i, 0)),
        out_shape=jax.ShapeDtypeStruct((b * n, 128), jnp.float32),
    )(y1, scale, shift, w2t_pad, b2_pad)
    return out[:, :nc].reshape(b, n, nc)


def kernel(xyz, points, params):
    l0_xyz = jnp.transpose(xyz, (0, 2, 1))
    l0_points = jnp.transpose(points, (0, 2, 1))
    l1_xyz, l1_points = _sa_layer(l0_xyz, l0_points, 1024, 0.1, 32,
                                  params['sa1'])
    l2_xyz, l2_points = _sa_layer(l1_xyz, l1_points, 256, 0.2, 32,
                                  params['sa2'])
    l3_xyz, l3_points = _sa_layer(l2_xyz, l2_points, 64, 0.4, 32,
                                  params['sa3'])
    l4_xyz, l4_points = _sa_layer(l3_xyz, l3_points, 16, 0.8, 32,
                                  params['sa4'])
    l3_points = _fp_layer(l3_xyz, l4_xyz, l3_points, l4_points, params['fp4'])
    l2_points = _fp_layer(l2_xyz, l3_xyz, l2_points, l3_points, params['fp3'])
    l1_points = _fp_layer(l1_xyz, l2_xyz, l1_points, l2_points, params['fp2'])
    l0_points = _fp_layer(l0_xyz, l1_xyz, l0_points, l1_points, params['fp1'])
    return _head(l0_points, params)


# speedup vs baseline: 1.3478x; 1.3478x over previous
import functools

import jax
import jax.numpy as jnp
from jax.experimental import pallas as pl


def _square_distance(src, dst):
    return (jnp.sum(src ** 2, -1)[:, :, None]
            + jnp.sum(dst ** 2, -1)[:, None, :]
            - 2.0 * jnp.einsum('bnc,bmc->bnm', src, dst))


def _pd_kernel(src_ref, dstT_ref, o_ref, *, radius, n_sentinel):
    a = src_ref[0]
    xT = dstT_ref[0]
    s2 = (a[:, 0:1] * a[:, 0:1] + a[:, 1:2] * a[:, 1:2]) + a[:, 2:3] * a[:, 2:3]
    x2 = (xT[0:1] * xT[0:1] + xT[1:2] * xT[1:2]) + xT[2:3] * xT[2:3]
    e = jax.lax.dot_general(
        a.astype(jnp.bfloat16), xT.astype(jnp.bfloat16),
        (((1,), (0,)), ((), ())), preferred_element_type=jnp.float32)
    d2 = (s2 + x2) - 2.0 * e
    if radius is None:
        o_ref[0] = d2
    else:
        iota = jax.lax.broadcasted_iota(jnp.int32, d2.shape, 1)
        o_ref[0] = jnp.where(d2 > radius ** 2, n_sentinel, iota)


def _pairdist(src, dst, radius=None):
    b, s, _ = src.shape
    n = dst.shape[1]
    dst_t = jnp.transpose(dst, (0, 2, 1))
    r = min(s, 256)
    out_dtype = jnp.float32 if radius is None else jnp.int32
    return pl.pallas_call(
        functools.partial(_pd_kernel, radius=radius, n_sentinel=n),
        grid=(b, s // r),
        in_specs=[
            pl.BlockSpec((1, r, 3), lambda i, j: (i, j, 0)),
            pl.BlockSpec((1, 3, n), lambda i, j: (i, 0, 0)),
        ],
        out_specs=pl.BlockSpec((1, r, n), lambda i, j: (i, j, 0)),
        out_shape=jax.ShapeDtypeStruct((b, s, n), out_dtype),
    )(src, dst_t)


def _index_points(points, idx):
    return jax.vmap(lambda p, i: p[i])(points, idx)


def _fps_kernel(x_ref, y_ref, z_ref, out_ref, *, npoint, n, b):
    x = x_ref[...]
    y = y_ref[...]
    z = z_ref[...]
    iota_n = jax.lax.broadcasted_iota(jnp.int32, (b, n), 1)
    iota_s = jax.lax.broadcasted_iota(jnp.int32, (b, npoint), 1)

    out_ref[...] = jnp.zeros((b, npoint), jnp.int32)

    def body(i, state):
        distance, farthest = state
        far_s = jnp.broadcast_to(farthest, (b, npoint))
        out_ref[...] = jnp.where(iota_s == i, far_s, out_ref[...])
        sel = iota_n == jnp.broadcast_to(farthest, (b, n))
        cx = jnp.sum(jnp.where(sel, x, 0.0), axis=1, keepdims=True)
        cy = jnp.sum(jnp.where(sel, y, 0.0), axis=1, keepdims=True)
        cz = jnp.sum(jnp.where(sel, z, 0.0), axis=1, keepdims=True)
        d = (x - cx) ** 2 + (y - cy) ** 2 + (z - cz) ** 2
        distance = jnp.minimum(distance, d)
        m = jnp.max(distance, axis=1, keepdims=True)
        farthest = jnp.min(jnp.where(distance == m, iota_n, n), axis=1,
                           keepdims=True).astype(jnp.int32)
        return distance, farthest

    dist0 = jnp.full((b, n), 1e10, jnp.float32)
    far0 = jnp.zeros((b, 1), jnp.int32)
    jax.lax.fori_loop(0, npoint, body, (dist0, far0))


def _farthest_point_sample(xyz, npoint):
    xyz = jax.lax.stop_gradient(xyz)
    b, n, _ = xyz.shape

    def body(i, state):
        centroids, distance, farthest = state
        centroids = centroids.at[:, i].set(farthest)
        centroid = _index_points(xyz, farthest[:, None])
        dist = jnp.sum((xyz - centroid) ** 2, -1)
        distance = jnp.minimum(distance, dist)
        farthest = jnp.argmax(distance, -1).astype(jnp.int32)
        return (centroids, distance, farthest)

    centroids = jnp.zeros((b, npoint), dtype=jnp.int32)
    distance = jnp.full((b, n), 1e10, dtype=xyz.dtype)
    farthest = jnp.zeros((b,), dtype=jnp.int32)
    centroids, _, _ = jax.lax.fori_loop(0, npoint, body,
                                        (centroids, distance, farthest))
    return centroids


def _query_ball_point(radius, nsample, xyz, new_xyz):
    xyz = jax.lax.stop_gradient(xyz)
    new_xyz = jax.lax.stop_gradient(new_xyz)
    n = xyz.shape[1]
    sqrdists = _square_distance(new_xyz, xyz)
    group_idx = jnp.broadcast_to(
        jnp.arange(n, dtype=jnp.int32)[None, None, :], sqrdists.shape)
    group_idx = jnp.where(sqrdists > radius ** 2, n, group_idx)
    group_idx = -jax.lax.top_k(-group_idx, nsample)[0]
    group_first = jnp.broadcast_to(group_idx[:, :, :1], group_idx.shape)
    group_idx = jnp.where(group_idx == n, group_first, group_idx)
    return group_idx


def _batchnorm(x, gamma, beta, axes):
    mean = jnp.mean(x, axis=axes, keepdims=True)
    var = jnp.mean((x - mean) ** 2, axis=axes, keepdims=True)
    return gamma * (x - mean) / jnp.sqrt(var + 1e-5) + beta


def _fps_pallas(x, y, z, npoint):
    b, n = x.shape
    return pl.pallas_call(
        functools.partial(_fps_kernel, npoint=npoint, n=n, b=b),
        out_shape=jax.ShapeDtypeStruct((b, npoint), jnp.int32),
    )(x, y, z)


def _sa_layer(xyz, points, npoint, radius, nsample, layers, fps_idx=None):
    if fps_idx is None:
        fps_idx = _farthest_point_sample(xyz, npoint)
    new_xyz = _index_points(xyz, fps_idx)
    idx = _query_ball_point(radius, nsample, xyz, new_xyz)
    grouped_xyz = _index_points(xyz, idx)
    grouped_xyz_norm = grouped_xyz - new_xyz[:, :, None, :]
    grouped_points = _index_points(points, idx)
    h = jnp.concatenate([grouped_xyz_norm, grouped_points], axis=-1)
    for lyr in layers:
        h = h @ lyr['W'].T + lyr['b']
        h = _batchnorm(h, lyr['gamma'], lyr['beta'], (0, 1, 2))
        h = jax.nn.relu(h)
    new_points = jnp.max(h, axis=2)
    return new_xyz, new_points


def _fp_layer(xyz1, xyz2, points1, points2, layers):
    b, n, _ = xyz1.shape
    s = xyz2.shape[1]
    if s == 1:
        interpolated = jnp.broadcast_to(points2, (b, n, points2.shape[-1]))
    else:
        dists = _square_distance(xyz1, xyz2)
        idx = jnp.argsort(dists, axis=-1)[:, :, :3]
        d3 = jnp.take_along_axis(dists, idx, axis=-1)
        dist_recip = 1.0 / (d3 + 1e-8)
        norm = jnp.sum(dist_recip, axis=2, keepdims=True)
        weight = dist_recip / norm
        interpolated = jnp.sum(
            _index_points(points2, idx) * weight[..., None], axis=2)
    if points1 is not None:
        h = jnp.concatenate([points1, interpolated], axis=-1)
    else:
        h = interpolated
    for lyr in layers:
        h = h @ lyr['W'].T + lyr['b']
        h = _batchnorm(h, lyr['gamma'], lyr['beta'], (0, 1))
        h = jax.nn.relu(h)
    return h


_ROWS = 1024


def _mm_bias_kernel(x_ref, w_ref, b_ref, o_ref):
    o_ref[...] = (
        jnp.dot(x_ref[...], w_ref[...],
                preferred_element_type=jnp.float32) + b_ref[...])


def _bn_relu_mm_kernel(y_ref, scale_ref, shift_ref, w_ref, b_ref, o_ref):
    h = jax.nn.relu(y_ref[...] * scale_ref[...] + shift_ref[...])
    o_ref[...] = (
        jnp.dot(h, w_ref[...], preferred_element_type=jnp.float32)
        + b_ref[...])


def _head(h, params):
    b, n, c = h.shape
    x = h.reshape(b * n, c)
    w1t = params['conv1']['W'].T
    b1 = params['conv1']['b'][None, :]
    y1 = pl.pallas_call(
        _mm_bias_kernel,
        grid=(b * n // _ROWS,),
        in_specs=[
            pl.BlockSpec((_ROWS, c), lambda i: (i, 0)),
            pl.BlockSpec((c, c), lambda i: (0, 0)),
            pl.BlockSpec((1, c), lambda i: (0, 0)),
        ],
        out_specs=pl.BlockSpec((_ROWS, c), lambda i: (i, 0)),
        out_shape=jax.ShapeDtypeStruct((b * n, c), jnp.float32),
    )(x, w1t, b1)

    mean = jnp.mean(y1, axis=0, keepdims=True)
    var = jnp.mean((y1 - mean) ** 2, axis=0, keepdims=True)
    gamma = params['bn1']['gamma'][None, :]
    beta = params['bn1']['beta'][None, :]
    scale = gamma / jnp.sqrt(var + 1e-5)
    shift = beta - mean * scale

    w2 = params['conv2']['W']
    nc = w2.shape[0]
    w2t_pad = jnp.zeros((c, 128), jnp.float32).at[:, :nc].set(w2.T)
    b2_pad = jnp.zeros((1, 128), jnp.float32).at[0, :nc].set(
        params['conv2']['b'])
    out = pl.pallas_call(
        _bn_relu_mm_kernel,
        grid=(b * n // _ROWS,),
        in_specs=[
            pl.BlockSpec((_ROWS, c), lambda i: (i, 0)),
            pl.BlockSpec((1, c), lambda i: (0, 0)),
            pl.BlockSpec((1, c), lambda i: (0, 0)),
            pl.BlockSpec((c, 128), lambda i: (0, 0)),
            pl.BlockSpec((1, 128), lambda i: (0, 0)),
        ],
        out_specs=pl.BlockSpec((_ROWS, 128), lambda i: (i, 0)),
        out_shape=jax.ShapeDtypeStruct((b * n, 128), jnp.float32),
    )(y1, scale, shift, w2t_pad, b2_pad)
    return out[:, :nc].reshape(b, n, nc)


def kernel(xyz, points, params):
    l0_xyz = jnp.transpose(xyz, (0, 2, 1))
    l0_points = jnp.transpose(points, (0, 2, 1))
    fps1 = _fps_pallas(xyz[:, 0, :], xyz[:, 1, :], xyz[:, 2, :], 1024)
    l1_xyz, l1_points = _sa_layer(l0_xyz, l0_points, 1024, 0.1, 32,
                                  params['sa1'], fps_idx=fps1)
    l2_xyz, l2_points = _sa_layer(l1_xyz, l1_points, 256, 0.2, 32,
                                  params['sa2'])
    l3_xyz, l3_points = _sa_layer(l2_xyz, l2_points, 64, 0.4, 32,
                                  params['sa3'])
    l4_xyz, l4_points = _sa_layer(l3_xyz, l3_points, 16, 0.8, 32,
                                  params['sa4'])
    l3_points = _fp_layer(l3_xyz, l4_xyz, l3_points, l4_points, params['fp4'])
    l2_points = _fp_layer(l2_xyz, l3_xyz, l2_points, l3_points, params['fp3'])
    l1_points = _fp_layer(l1_xyz, l2_xyz, l1_points, l2_points, params['fp2'])
    l0_points = _fp_layer(l0_xyz, l1_xyz, l0_points, l1_points, params['fp1'])
    return _head(l0_points, params)
